# baseline (speedup 1.0000x reference)
import functools

import jax
import jax.numpy as jnp
from jax import lax
from jax.experimental import pallas as pl
from jax.experimental.pallas import tpu as pltpu
from jax.experimental.pallas import tpu_sc as plsc

_NC = 2
_NT = 16
_GU = 5


def _sc_body(l, s, v, d, xt, tbl, out, idx_v, vals0, vals1, gs0, gs1, ws0, ws1,
             row_sh):
    core = lax.axis_index("c")
    tile = lax.axis_index("s")
    dpc = d // _NC
    q = s // 4
    upt = l * 4 // _NT
    gpc = upt // _GU
    n = upt * q
    gsz = _GU * q
    vals = (vals0, vals1)
    gsem = (gs0, gs1)
    wsem = (ws0, ws1)

    pltpu.async_copy(xt.at[pl.ds(tile * n, n)], idx_v, gs0).wait()

    chunk = (v // _NT) & ~7
    rem = v - chunk * _NT
    c0 = tile * chunk

    def gather(g, b):
        return pltpu.make_async_copy(
            row_sh.at[idx_v.at[pl.ds(g * gsz, gsz)]], vals[b], gsem[b])

    def drain_writes(b):
        for _ in range(_GU):
            pltpu.make_async_copy(vals[b].at[pl.ds(0, q)],
                                  out.at[pl.ds(0, q)], wsem[b]).wait()

    @pl.loop(0, dpc)
    def _(c):
        row = core * dpc + c
        pltpu.sync_copy(tbl.at[pl.ds(row * v + c0, chunk)],
                        row_sh.at[pl.ds(c0, chunk)])
        if rem:
            @pl.when(tile == 0)
            def _():
                pltpu.sync_copy(tbl.at[pl.ds(row * v + chunk * _NT, rem)],
                                row_sh.at[pl.ds(chunk * _NT, rem)])
        plsc.subcore_barrier()

        gather(0, 0).start()
        for g in range(gpc):
            b = g % 2
            if g + 1 < gpc:
                if g >= 1:
                    drain_writes((g + 1) % 2)
                gather(g + 1, (g + 1) % 2).start()
            gather(g, b).wait()
            for k in range(_GU):
                u = tile * upt + g * _GU + k
                li = u // 4
                sq = lax.rem(u, 4)
                off = (li * d + row) * s + sq * q
                pltpu.async_copy(vals[b].at[pl.ds(k * q, q)],
                                 out.at[pl.ds(off, q)], wsem[b])
        drain_writes(0)
        drain_writes(1)
        plsc.subcore_barrier()


@functools.partial(jax.jit, static_argnums=(2, 3, 4, 5))
def _impl(xt, tblt, l, s, v, d):
    mesh = plsc.VectorSubcoreMesh(core_axis_name="c", subcore_axis_name="s")
    n = l * s // _NT
    k = pl.kernel(
        functools.partial(_sc_body, l, s, v, d),
        out_type=jax.ShapeDtypeStruct((l * d * s,), jnp.float32),
        mesh=mesh,
        scratch_types=(
            pltpu.VMEM((n,), jnp.int32),
            pltpu.VMEM((_GU * s // 4,), jnp.float32),
            pltpu.VMEM((_GU * s // 4,), jnp.float32),
            pltpu.SemaphoreType.DMA,
            pltpu.SemaphoreType.DMA,
            pltpu.SemaphoreType.DMA,
            pltpu.SemaphoreType.DMA,
            pltpu.VMEM_SHARED((v,), jnp.float32),
        ),
        compiler_params=pltpu.CompilerParams(use_tc_tiling_on_sc=False),
    )
    return k(xt, tblt)


def kernel(x, table):
    s, l = x.shape
    v, d = table.shape
    xt = x.T.astype(jnp.int32).reshape(-1)
    tblt = table.T.reshape(-1)
    out1 = _impl(xt, tblt, l, s, v, d)
    return out1.reshape(l, d, s).transpose(2, 0, 1)


# speedup vs baseline: 4.8072x; 4.8072x over previous
import functools

import jax
import jax.numpy as jnp
from jax import lax
from jax.experimental import pallas as pl
from jax.experimental.pallas import tpu as pltpu
from jax.experimental.pallas import tpu_sc as plsc

_NC = 2
_NS = 16
_NW = _NC * _NS
_CH = 512


def _body(nch, d, idx_hbm, table_hbm, out_hbm, idx_v, rows0, rows1, sem0, sem1):
    wid = lax.axis_index("s") * _NC + lax.axis_index("c")
    pltpu.sync_copy(idx_hbm.at[wid], idx_v)
    base = wid * (nch * _CH)

    pltpu.async_copy(table_hbm.at[idx_v.at[0]], rows0, sem0)
    pltpu.async_copy(table_hbm.at[idx_v.at[1]], rows1, sem1)

    @pl.loop(0, nch - 2, step=2)
    def _(g):
        pltpu.make_async_copy(table_hbm.at[idx_v.at[g]], rows0, sem0).wait()
        pltpu.sync_copy(rows0, out_hbm.at[pl.ds(base + g * _CH, _CH)])
        pltpu.async_copy(table_hbm.at[idx_v.at[g + 2]], rows0, sem0)

        pltpu.make_async_copy(table_hbm.at[idx_v.at[g + 1]], rows1, sem1).wait()
        pltpu.sync_copy(rows1, out_hbm.at[pl.ds(base + (g + 1) * _CH, _CH)])
        pltpu.async_copy(table_hbm.at[idx_v.at[g + 3]], rows1, sem1)

    pltpu.make_async_copy(table_hbm.at[idx_v.at[nch - 2]], rows0, sem0).wait()
    pltpu.sync_copy(rows0, out_hbm.at[pl.ds(base + (nch - 2) * _CH, _CH)])
    pltpu.make_async_copy(table_hbm.at[idx_v.at[nch - 1]], rows1, sem1).wait()
    pltpu.sync_copy(rows1, out_hbm.at[pl.ds(base + (nch - 1) * _CH, _CH)])


@functools.partial(jax.jit, static_argnums=(2, 3, 4))
def _gather(idx, table, b, nch, d):
    mesh = plsc.VectorSubcoreMesh(core_axis_name="c", subcore_axis_name="s")
    k = pl.kernel(
        functools.partial(_body, nch, d),
        out_type=jax.ShapeDtypeStruct((b, d), jnp.float32),
        mesh=mesh,
        scratch_types=[
            pltpu.VMEM((nch, _CH), jnp.int32),
            pltpu.VMEM((_CH, d), jnp.float32),
            pltpu.VMEM((_CH, d), jnp.float32),
            pltpu.SemaphoreType.DMA,
            pltpu.SemaphoreType.DMA,
        ],
        compiler_params=pltpu.CompilerParams(use_tc_tiling_on_sc=False),
    )
    return k(idx, table)


def kernel(x, table):
    s, l = x.shape
    v, d = table.shape
    b = s * l
    nch = b // (_NW * _CH)
    idx = x.astype(jnp.int32).reshape(_NW, nch, _CH)
    out = _gather(idx, table, b, nch, d)
    return out.reshape(s, l, d)


# speedup vs baseline: 4.8115x; 1.0009x over previous
import functools

import jax
import jax.numpy as jnp
from jax import lax
from jax.experimental import pallas as pl
from jax.experimental.pallas import tpu as pltpu
from jax.experimental.pallas import tpu_sc as plsc

_NC = 2
_NS = 16
_NW = _NC * _NS
_CH = 320
_NBUF = 4


def _body(nch, d, idx_hbm, table_hbm, out_hbm, idx_v,
          r0, r1, r2, r3, gs0, gs1, gs2, gs3, ws0, ws1, ws2, ws3):
    rows = (r0, r1, r2, r3)
    gsem = (gs0, gs1, gs2, gs3)
    wsem = (ws0, ws1, ws2, ws3)
    wid = lax.axis_index("s") * _NC + lax.axis_index("c")
    pltpu.sync_copy(idx_hbm.at[wid], idx_v)
    base = wid * (nch * _CH)

    def issue_gather(c, b):
        pltpu.async_copy(table_hbm.at[idx_v.at[c]], rows[b], gsem[b])

    def issue_wb(c, b):
        pltpu.async_copy(rows[b], out_hbm.at[pl.ds(base + c * _CH, _CH)],
                         wsem[b])

    def wait_g(b):
        pltpu.make_async_copy(table_hbm.at[idx_v.at[0]], rows[b], gsem[b]).wait()

    def wait_w(b):
        pltpu.make_async_copy(rows[b], out_hbm.at[pl.ds(base, _CH)],
                              wsem[b]).wait()

    issue_gather(0, 0)
    issue_gather(1, 1)
    issue_gather(2, 2)
    wait_g(0)
    issue_wb(0, 0)
    issue_gather(3, 3)
    wait_g(1)
    issue_wb(1, 1)

    @pl.loop(2, nch - 2, step=4)
    def _(g):
        for k in range(4):
            j = g + k
            b = (2 + k) % 4
            bn = (b + 2) % 4
            wait_w(bn)
            issue_gather(j + 2, bn)
            wait_g(b)
            issue_wb(j, b)

    for j in (nch - 2, nch - 1):
        b = j % 4
        wait_g(b)
        issue_wb(j, b)
    for b in range(_NBUF):
        wait_w(b)


@functools.partial(jax.jit, static_argnums=(2, 3, 4))
def _gather(idx, table, b, nch, d):
    mesh = plsc.VectorSubcoreMesh(core_axis_name="c", subcore_axis_name="s")
    k = pl.kernel(
        functools.partial(_body, nch, d),
        out_type=jax.ShapeDtypeStruct((b, d), jnp.float32),
        mesh=mesh,
        scratch_types=(
            [pltpu.VMEM((nch, _CH), jnp.int32)]
            + [pltpu.VMEM((_CH, d), jnp.float32)] * _NBUF
            + [pltpu.SemaphoreType.DMA] * (2 * _NBUF)
        ),
        compiler_params=pltpu.CompilerParams(use_tc_tiling_on_sc=False),
    )
    return k(idx, table)


def kernel(x, table):
    s, l = x.shape
    v, d = table.shape
    b = s * l
    nch = b // (_NW * _CH)
    idx = x.astype(jnp.int32).reshape(_NW, nch, _CH)
    out = _gather(idx, table, b, nch, d)
    return out.reshape(s, l, d)
